# baseline (speedup 1.0000x reference)
import functools

import jax
import jax.numpy as jnp
from jax import lax
from jax.experimental import pallas as pl
from jax.experimental.pallas import tpu as pltpu
from jax.experimental.pallas import tpu_sc as plsc

VOCAB = 100000
MAXLEN = 200
EMBED = 64
BATCH = 4096

NC = 2
NS = 16
NW = NC * NS
BW = BATCH // NW
NTILE = BATCH // 128
NE = EMBED // 8
TILE = 8 * 128
OUT_ELEMS = BATCH * MAXLEN * EMBED
NBUF = 2


def _body(xt_hbm, tok_hbm, pos_hbm, out_hbm,
          pos_v, idx0, idx1, g0, g1, t0, t1, sg0, sg1, so0, so1):
    idx_v = (idx0, idx1)
    gbuf = (g0, g1)
    tbuf = (t0, t1)
    sem_g = (sg0, sg1)
    sem_o = (so0, so1)

    w = lax.axis_index("s") * NC + lax.axis_index("c")
    bbase = w * BW

    pltpu.sync_copy(pos_hbm, pos_v)

    iota16 = lax.iota(jnp.int32, 16)
    rots = [(iota16 + i) & 15 for i in range(16)]

    def gather_desc(b):
        return pltpu.make_async_copy(tok_hbm.at[idx_v[b]], gbuf[b], sem_g[b])

    def start_gather(m, b):
        pltpu.sync_copy(xt_hbm.at[m, pl.ds(bbase, BW)], idx_v[b])
        gather_desc(b).start()

    def store_descs(m, b):
        return [
            pltpu.make_async_copy(
                tbuf[b].at[pl.ds(e * TILE, TILE)],
                out_hbm.at[pl.ds(((m * NE + e) * NTILE + w) * TILE, TILE)],
                sem_o[b])
            for e in range(NE)
        ]

    start_gather(0, 0)
    start_gather(1, 1)

    def step(m, b):
        gather_desc(b).wait()

        @pl.when(m >= NBUF)
        def _():
            for d in store_descs(m - NBUF, b):
                d.wait()

        for j in range(EMBED // 16):
            prj = pos_v[m, pl.ds(16 * j, 16)]
            prot = [
                prj.at[rots[i]].get(mode="promise_in_bounds") if i else prj
                for i in range(16)
            ]

            @plsc.parallel_loop(0, BW // 16, unroll=2)
            def _blk(a):
                rowv = iota16 + a * 16
                cols = [rots[i] + 16 * j for i in range(16)]
                vs = [plsc.load_gather(gbuf[b], [rowv, cols[i]])
                      for i in range(16)]
                for i in range(16):
                    plsc.store_scatter(
                        tbuf[b], [cols[i] * 128 + rowv], vs[i] + prot[i])

        for d in store_descs(m, b):
            d.start()

        @pl.when(m + NBUF < MAXLEN)
        def _():
            start_gather(m + NBUF, b)

    def group(g, carry):
        for b in range(NBUF):
            step(g * NBUF + b, b)
        return carry

    lax.fori_loop(0, MAXLEN // NBUF, group, 0)

    for m in (MAXLEN - 2, MAXLEN - 1):
        for d in store_descs(m, m % NBUF):
            d.wait()


@jax.jit
def kernel(x, token_table, pos_table):
    mesh = plsc.VectorSubcoreMesh(
        core_axis_name="c", subcore_axis_name="s",
        num_cores=NC, num_subcores=NS)
    f = pl.kernel(
        _body,
        out_type=jax.ShapeDtypeStruct((OUT_ELEMS,), jnp.float32),
        mesh=mesh,
        scratch_types=(
            [pltpu.VMEM((MAXLEN, EMBED), jnp.float32)]
            + [pltpu.VMEM((BW,), jnp.int32)] * NBUF
            + [pltpu.VMEM((BW, EMBED), jnp.float32)] * NBUF
            + [pltpu.VMEM((EMBED * BW,), jnp.float32)] * NBUF
            + [pltpu.SemaphoreType.DMA] * (2 * NBUF)
        ),
        compiler_params=pltpu.CompilerParams(
            use_tc_tiling_on_sc=False, needs_layout_passes=False),
    )
    xt = jnp.transpose(x.astype(jnp.int32))
    out1d = f(xt, token_table, pos_table)
    arr5 = out1d.reshape(MAXLEN, NE, NTILE, 8, 128)
    return arr5.transpose(2, 4, 0, 1, 3).reshape(BATCH, MAXLEN, EMBED)


# speedup vs baseline: 1.3504x; 1.3504x over previous
import functools

import jax
import jax.numpy as jnp
from jax import lax
from jax.experimental import pallas as pl
from jax.experimental.pallas import tpu as pltpu
from jax.experimental.pallas import tpu_sc as plsc

VOCAB = 100000
MAXLEN = 200
EMBED = 64
BATCH = 4096

NC = 2
NS = 16
NW = NC * NS
BW = BATCH // NW
NTILE = BATCH // 128
NE = EMBED // 8
TILE = 8 * 128
OUT_ELEMS = BATCH * MAXLEN * EMBED
NBUF = 2


def _body(xt_hbm, tok_hbm, pos_hbm, out_hbm,
          pos_v, idx0, idx1, g0, g1, t0, t1, sg0, sg1, so0, so1):
    idx_v = (idx0, idx1)
    gbuf = (g0, g1)
    tbuf = (t0, t1)
    sem_g = (sg0, sg1)
    sem_o = (so0, so1)

    w = lax.axis_index("s") * NC + lax.axis_index("c")
    bbase = w * BW

    pltpu.sync_copy(pos_hbm, pos_v)

    iota16 = lax.iota(jnp.int32, 16)
    rots = [(iota16 + i) & 15 for i in range(16)]

    def gather_desc(b):
        return pltpu.make_async_copy(tok_hbm.at[idx_v[b]], gbuf[b], sem_g[b])

    def start_gather(m, b):
        pltpu.sync_copy(xt_hbm.at[m, pl.ds(bbase, BW)], idx_v[b])
        gather_desc(b).start()

    def store_descs(m, b):
        return [
            pltpu.make_async_copy(
                tbuf[b].at[pl.ds(e * TILE, TILE)],
                out_hbm.at[pl.ds(((m * NE + e) * NTILE + w) * TILE, TILE)],
                sem_o[b])
            for e in range(NE)
        ]

    start_gather(0, 0)
    start_gather(1, 1)

    def step(m, b):
        gather_desc(b).wait()

        @pl.when(m >= NBUF)
        def _():
            for d in store_descs(m - NBUF, b):
                d.wait()

        for j in range(EMBED // 16):
            prj = pos_v[m, pl.ds(16 * j, 16)]
            prot = [
                prj.at[rots[i]].get(mode="promise_in_bounds") if i else prj
                for i in range(16)
            ]

            cols = [rots[i] + 16 * j for i in range(16)]

            @plsc.parallel_loop(0, BW // 16)
            def _blk(a):
                rowv = iota16 + a * 16
                vs = [plsc.load_gather(gbuf[b], [rowv, cols[i]])
                      for i in range(16)]
                for i in range(16):
                    plsc.store_scatter(
                        tbuf[b], [cols[i] * 128 + rowv], vs[i] + prot[i])

        for d in store_descs(m, b):
            d.start()

        @pl.when(m + NBUF < MAXLEN)
        def _():
            start_gather(m + NBUF, b)

    def group(g, carry):
        for b in range(NBUF):
            step(g * NBUF + b, b)
        return carry

    lax.fori_loop(0, MAXLEN // NBUF, group, 0)

    for m in (MAXLEN - 2, MAXLEN - 1):
        for d in store_descs(m, m % NBUF):
            d.wait()


@jax.jit
def kernel(x, token_table, pos_table):
    mesh = plsc.VectorSubcoreMesh(
        core_axis_name="c", subcore_axis_name="s",
        num_cores=NC, num_subcores=NS)
    f = pl.kernel(
        _body,
        out_type=jax.ShapeDtypeStruct((OUT_ELEMS,), jnp.float32),
        mesh=mesh,
        scratch_types=(
            [pltpu.VMEM((MAXLEN, EMBED), jnp.float32)]
            + [pltpu.VMEM((BW,), jnp.int32)] * NBUF
            + [pltpu.VMEM((BW, EMBED), jnp.float32)] * NBUF
            + [pltpu.VMEM((EMBED * BW,), jnp.float32)] * NBUF
            + [pltpu.SemaphoreType.DMA] * (2 * NBUF)
        ),
        compiler_params=pltpu.CompilerParams(
            use_tc_tiling_on_sc=False, needs_layout_passes=False),
    )
    xt = jnp.transpose(x.astype(jnp.int32))
    out1d = f(xt, token_table, pos_table)
    arr5 = out1d.reshape(MAXLEN, NE, NTILE, 8, 128)
    return arr5.transpose(2, 4, 0, 1, 3).reshape(BATCH, MAXLEN, EMBED)


# speedup vs baseline: 1.6578x; 1.2276x over previous
import functools

import jax
import jax.numpy as jnp
from jax import lax
from jax.experimental import pallas as pl
from jax.experimental.pallas import tpu as pltpu
from jax.experimental.pallas import tpu_sc as plsc

VOCAB = 100000
MAXLEN = 200
EMBED = 64
BATCH = 4096

NC = 2
NS = 16
NW = NC * NS
BW = BATCH // NW
NTILE = BATCH // 128
NE = EMBED // 8
TILE = 8 * 128
OUT_ELEMS = BATCH * MAXLEN * EMBED
NBUF = 2


def _body(xt_hbm, tok_hbm, pos_hbm, out_hbm,
          pos_v, idx_all, g0, g1, t0, t1, sg0, sg1, so0, so1):
    gbuf = (g0, g1)
    tbuf = (t0, t1)
    sem_g = (sg0, sg1)
    sem_o = (so0, so1)

    w = lax.axis_index("s") * NC + lax.axis_index("c")
    bbase = w * BW

    pltpu.sync_copy(pos_hbm, pos_v)
    pltpu.sync_copy(xt_hbm.at[:, pl.ds(bbase, BW)], idx_all)

    iota16 = lax.iota(jnp.int32, 16)
    rots = [(iota16 + i) & 15 for i in range(16)]

    def gather_desc(m, b):
        return pltpu.make_async_copy(
            tok_hbm.at[idx_all.at[m]], gbuf[b], sem_g[b])

    def start_gather(m, b):
        gather_desc(m, b).start()

    def store_descs(m, b):
        return [
            pltpu.make_async_copy(
                tbuf[b].at[pl.ds(e * TILE, TILE)],
                out_hbm.at[pl.ds(((m * NE + e) * NTILE + w) * TILE, TILE)],
                sem_o[b])
            for e in range(NE)
        ]

    start_gather(0, 0)
    start_gather(1, 1)

    def step(m, b):
        gather_desc(m, b).wait()

        @pl.when(m >= NBUF)
        def _():
            for d in store_descs(m - NBUF, b):
                d.wait()

        for j in range(EMBED // 16):
            prj = pos_v[m, pl.ds(16 * j, 16)]
            prot = [
                prj.at[rots[i]].get(mode="promise_in_bounds") if i else prj
                for i in range(16)
            ]

            cols = [rots[i] + 16 * j for i in range(16)]

            @plsc.parallel_loop(0, BW // 16)
            def _blk(a):
                rowv = iota16 + a * 16
                vs = [plsc.load_gather(gbuf[b], [rowv, cols[i]])
                      for i in range(16)]
                for i in range(16):
                    plsc.store_scatter(
                        tbuf[b], [cols[i] * 128 + rowv], vs[i] + prot[i])

        for d in store_descs(m, b):
            d.start()

        @pl.when(m + NBUF < MAXLEN)
        def _():
            start_gather(m + NBUF, b)

    def group(g, carry):
        for b in range(NBUF):
            step(g * NBUF + b, b)
        return carry

    lax.fori_loop(0, MAXLEN // NBUF, group, 0)

    for m in (MAXLEN - 2, MAXLEN - 1):
        for d in store_descs(m, m % NBUF):
            d.wait()


@jax.jit
def kernel(x, token_table, pos_table):
    mesh = plsc.VectorSubcoreMesh(
        core_axis_name="c", subcore_axis_name="s",
        num_cores=NC, num_subcores=NS)
    f = pl.kernel(
        _body,
        out_type=jax.ShapeDtypeStruct((OUT_ELEMS,), jnp.float32),
        mesh=mesh,
        scratch_types=(
            [pltpu.VMEM((MAXLEN, EMBED), jnp.float32)]
            + [pltpu.VMEM((MAXLEN, BW), jnp.int32)]
            + [pltpu.VMEM((BW, EMBED), jnp.float32)] * NBUF
            + [pltpu.VMEM((EMBED * BW,), jnp.float32)] * NBUF
            + [pltpu.SemaphoreType.DMA] * (2 * NBUF)
        ),
        compiler_params=pltpu.CompilerParams(
            use_tc_tiling_on_sc=False, needs_layout_passes=False),
    )
    xt = jnp.transpose(x.astype(jnp.int32))
    out1d = f(xt, token_table, pos_table)
    arr5 = out1d.reshape(MAXLEN, NE, NTILE, 8, 128)
    return arr5.transpose(2, 4, 0, 1, 3).reshape(BATCH, MAXLEN, EMBED)
